# baseline (speedup 1.0000x reference)
import math

import jax
import jax.numpy as jnp
from jax import lax
from jax.experimental import pallas as pl

_BLOCK_B = 512


def _fused_attn_kernel(z_ref, ctx_ref, embd_ref, out_ref):
    d = out_ref.shape[1]
    ctx = ctx_ref[...]
    embd_aug = embd_ref[...]
    scores = lax.dot_general(
        ctx, embd_aug[:, :d], (((1,), (1,)), ((), ())),
        preferred_element_type=jnp.float32,
    ) * (1.0 / math.sqrt(d))
    mf = (z_ref[...] > 0).astype(jnp.float32)
    row_max = jnp.max(scores, axis=1, keepdims=True)
    ex = jnp.exp(scores - row_max) * mf
    counts = jnp.maximum(jnp.sum(mf, axis=1, keepdims=True), 1.0)
    acc = jnp.dot(ex, embd_aug, preferred_element_type=jnp.float32)
    denom = acc[:, d:d + 1]
    denom = jnp.where(denom == 0.0, 1.0, denom)
    out_ref[...] = acc[:, :d] / (denom * counts)


def kernel(z_sparse, context_embedding, embd_weight):
    B, F = z_sparse.shape
    d = embd_weight.shape[1]
    embd_aug = jnp.concatenate(
        [embd_weight,
         jnp.ones((F, 1), jnp.float32),
         jnp.zeros((F, 127), jnp.float32)], axis=1)
    grid = (B // _BLOCK_B,)
    return pl.pallas_call(
        _fused_attn_kernel,
        grid=grid,
        in_specs=[
            pl.BlockSpec((_BLOCK_B, F), lambda i: (i, 0)),
            pl.BlockSpec((_BLOCK_B, d), lambda i: (i, 0)),
            pl.BlockSpec((F, d + 128), lambda i: (0, 0)),
        ],
        out_specs=pl.BlockSpec((_BLOCK_B, d), lambda i: (i, 0)),
        out_shape=jax.ShapeDtypeStruct((B, d), jnp.float32),
    )(z_sparse, context_embedding, embd_aug)


# speedup vs baseline: 1.3853x; 1.3853x over previous
import math

import jax
import jax.numpy as jnp
from jax import lax
from jax.experimental import pallas as pl

_BLOCK_B = 512


def _fused_attn_kernel(z_ref, ctx_ref, embd_ref, out_ref):
    d = embd_ref.shape[1]
    ctx = ctx_ref[...]
    embd = embd_ref[...]
    raw = lax.dot_general(
        ctx, embd, (((1,), (1,)), ((), ())),
        preferred_element_type=jnp.float32,
    )
    mf = (z_ref[...] > 0).astype(jnp.float32)
    row_max = jnp.max(raw, axis=1, keepdims=True)
    k = math.log2(math.e) / math.sqrt(d)
    ex = jnp.exp2((raw - row_max) * k) * mf
    denom = jnp.sum(ex, axis=1, keepdims=True)
    denom = jnp.where(denom == 0.0, 1.0, denom)
    counts = jnp.maximum(jnp.sum(mf, axis=1, keepdims=True), 1.0)
    acc = jnp.dot(ex, embd, preferred_element_type=jnp.float32)
    out_ref[...] = acc / (denom * counts)


def kernel(z_sparse, context_embedding, embd_weight):
    B, F = z_sparse.shape
    d = embd_weight.shape[1]
    grid = (B // _BLOCK_B,)
    return pl.pallas_call(
        _fused_attn_kernel,
        grid=grid,
        in_specs=[
            pl.BlockSpec((_BLOCK_B, F), lambda i: (i, 0)),
            pl.BlockSpec((_BLOCK_B, d), lambda i: (i, 0)),
            pl.BlockSpec((F, d), lambda i: (0, 0)),
        ],
        out_specs=pl.BlockSpec((_BLOCK_B, d), lambda i: (i, 0)),
        out_shape=jax.ShapeDtypeStruct((B, d), jnp.float32),
    )(z_sparse, context_embedding, embd_weight)


# speedup vs baseline: 1.4452x; 1.0432x over previous
import math

import jax
import jax.numpy as jnp
from jax import lax
from jax.experimental import pallas as pl
from jax.experimental.pallas import tpu as pltpu


def _fused_attn_kernel(z_hbm, ctx_ref, embd_ref, out_ref, z_vmem, sem):
    d = embd_ref.shape[1]
    copy = pltpu.make_async_copy(z_hbm, z_vmem, sem)
    copy.start()
    ctx = ctx_ref[...]
    embd = embd_ref[...]
    raw = lax.dot_general(
        ctx, embd, (((1,), (1,)), ((), ())),
        preferred_element_type=jnp.float32,
    )
    row_max = jnp.max(raw, axis=1, keepdims=True)
    copy.wait()
    mf = (z_vmem[...] > 0).astype(jnp.float32)
    k = math.log2(math.e) / math.sqrt(d)
    ex = jnp.exp2((raw - row_max) * k) * mf
    denom = jnp.sum(ex, axis=1, keepdims=True)
    denom = jnp.where(denom == 0.0, 1.0, denom)
    counts = jnp.maximum(jnp.sum(mf, axis=1, keepdims=True), 1.0)
    acc = jnp.dot(ex, embd, preferred_element_type=jnp.float32)
    out_ref[...] = acc / (denom * counts)


def kernel(z_sparse, context_embedding, embd_weight):
    B, F = z_sparse.shape
    d = embd_weight.shape[1]
    return pl.pallas_call(
        _fused_attn_kernel,
        in_specs=[
            pl.BlockSpec(memory_space=pltpu.MemorySpace.HBM),
            pl.BlockSpec((B, d), lambda: (0, 0)),
            pl.BlockSpec((F, d), lambda: (0, 0)),
        ],
        out_specs=pl.BlockSpec((B, d), lambda: (0, 0)),
        out_shape=jax.ShapeDtypeStruct((B, d), jnp.float32),
        scratch_shapes=[
            pltpu.VMEM((B, F), jnp.float32),
            pltpu.SemaphoreType.DMA,
        ],
    )(z_sparse, context_embedding, embd_weight)
